# baseline (speedup 1.0000x reference)
import functools

import jax
import jax.numpy as jnp
from jax import lax
from jax.experimental import pallas as pl
from jax.experimental.pallas import tpu as pltpu
from jax.experimental.pallas import tpu_sc as plsc

_ROW_START = 2
_NUM_ROWS = 19
_NUM_COLS = 128
_FLAT_OFF = _ROW_START * _NUM_COLS
_FLAT_LEN = _NUM_ROWS * _NUM_COLS

_MESH = plsc.VectorSubcoreMesh(core_axis_name="c", subcore_axis_name="s")


@functools.partial(
    pl.kernel,
    out_type=jax.ShapeDtypeStruct((_FLAT_LEN,), jnp.float32),
    mesh=_MESH,
)
def _sc_slice_copy(x_hbm, out_hbm):
    cid = lax.axis_index("c")
    sid = lax.axis_index("s")

    @pl.when(jnp.logical_and(cid == 0, sid == 0))
    def _():
        pltpu.sync_copy(x_hbm.at[pl.ds(_FLAT_OFF, _FLAT_LEN)], out_hbm)


def kernel(x):
    flat = _sc_slice_copy(x.reshape(-1))
    return flat.reshape(_NUM_ROWS, _NUM_COLS)


# speedup vs baseline: 1.0967x; 1.0967x over previous
import functools

import jax
import jax.numpy as jnp
from jax import lax
from jax.experimental import pallas as pl
from jax.experimental.pallas import tpu as pltpu
from jax.experimental.pallas import tpu_sc as plsc

_ROW_START = 2
_NUM_ROWS = 19
_NUM_COLS = 128
_FLAT_OFF = _ROW_START * _NUM_COLS
_FLAT_LEN = _NUM_ROWS * _NUM_COLS

_MESH = plsc.ScalarSubcoreMesh(axis_name="c", num_cores=2)


@functools.partial(
    pl.kernel,
    out_type=jax.ShapeDtypeStruct((_FLAT_LEN,), jnp.float32),
    mesh=_MESH,
)
def _sc_slice_copy(x_hbm, out_hbm):
    cid = lax.axis_index("c")

    @pl.when(cid == 0)
    def _():
        pltpu.sync_copy(x_hbm.at[pl.ds(_FLAT_OFF, _FLAT_LEN)], out_hbm)


def kernel(x):
    flat = _sc_slice_copy(x.reshape(-1))
    return flat.reshape(_NUM_ROWS, _NUM_COLS)


# speedup vs baseline: 1.1976x; 1.0921x over previous
import functools

import jax
import jax.numpy as jnp
from jax import lax
from jax.experimental import pallas as pl
from jax.experimental.pallas import tpu as pltpu
from jax.experimental.pallas import tpu_sc as plsc

_ROW_START = 2
_NUM_ROWS = 19
_NUM_COLS = 128
_FLAT_OFF = _ROW_START * _NUM_COLS
_FLAT_LEN = _NUM_ROWS * _NUM_COLS

_MESH = plsc.ScalarSubcoreMesh(axis_name="c", num_cores=1)


@functools.partial(
    pl.kernel,
    out_type=jax.ShapeDtypeStruct((_FLAT_LEN,), jnp.float32),
    mesh=_MESH,
)
def _sc_slice_copy(x_hbm, out_hbm):
    pltpu.sync_copy(x_hbm.at[pl.ds(_FLAT_OFF, _FLAT_LEN)], out_hbm)


def kernel(x):
    flat = _sc_slice_copy(x.reshape(-1))
    return flat.reshape(_NUM_ROWS, _NUM_COLS)
